# baseline (speedup 1.0000x reference)
import functools

import jax
import jax.numpy as jnp
from jax import lax
from jax.experimental import pallas as pl
from jax.experimental.pallas import tpu as pltpu
from jax.experimental.pallas import tpu_sc as plsc

NC = 2
NS = 16
NW = NC * NS
L = 16


def _round_up(x, m):
    return (x + m - 1) // m * m


@functools.partial(jax.jit, static_argnums=(4, 5))
def _gather_sc(p0_phys, p1_phys, idx0p, idx1p, pw0, n1p):
    B0 = p0_phys.shape[0]
    B1 = 8
    mesh = plsc.VectorSubcoreMesh(core_axis_name="c", subcore_axis_name="s")

    @functools.partial(
        pl.kernel,
        mesh=mesh,
        out_type=[
            jax.ShapeDtypeStruct((B0, NW * pw0), jnp.float32),
            jax.ShapeDtypeStruct((B1, n1p), jnp.float32),
        ],
        scratch_types=[
            pltpu.VMEM((pw0,), jnp.int32),
        ] + [pltpu.VMEM((pw0,), jnp.float32) for _ in range(B0)] + [
            pltpu.VMEM((n1p,), jnp.int32),
            pltpu.VMEM((n1p,), jnp.float32),
            pltpu.SemaphoreType.DMA,
            pltpu.SemaphoreType.DMA,
        ],
        compiler_params=pltpu.CompilerParams(use_tc_tiling_on_sc=False),
    )
    def body(p0, p1, i0, i1, out0, out1, idx_v, *rest):
        bufs = rest[:B0]
        idx1_v, vals1_v, gsem, ssem = rest[B0:]
        wid = lax.axis_index("c") * NS + lax.axis_index("s")
        base = wid * pw0
        pltpu.sync_copy(i0.at[pl.ds(base, pw0)], idx_v)

        def xform(s, _):
            v = idx_v[pl.ds(s * L, L)]
            idx_v[pl.ds(s * L, L)] = (
                (v & jnp.int32(~16383))
                | ((v & jnp.int32(1920)) << 3)
                | ((v >> 4) & jnp.int32(896))
                | (v & jnp.int32(127))
            )
            return 0

        lax.fori_loop(0, pw0 // L, xform, 0)

        gathers = [
            pltpu.async_copy(p0.at[b].at[idx_v], bufs[b], gsem)
            for b in range(B0)
        ]
        stores = []
        for b in range(B0):
            gathers[b].wait()
            stores.append(pltpu.async_copy(
                bufs[b], out0.at[b].at[pl.ds(base, pw0)], ssem))
        for st in stores:
            st.wait()

        @pl.when(wid < B1)
        def _small():
            pltpu.sync_copy(i1.at[pl.ds(0, n1p)], idx1_v)

            def xform1(s, _):
                v = idx1_v[pl.ds(s * L, L)]
                idx1_v[pl.ds(s * L, L)] = (
                    ((v & jnp.int32(~127)) << 3)
                    | (v & jnp.int32(127))
                ) + wid * 128
                return 0

            lax.fori_loop(0, n1p // L, xform1, 0)
            pltpu.async_copy(p1.at[0].at[idx1_v], vals1_v, gsem).wait()
            pltpu.sync_copy(vals1_v, out1.at[wid])

    return body(p0_phys, p1_phys, idx0p, idx1p)


def kernel(params_0, params_1, idx_0, idx_1):
    B0 = params_0.shape[0]
    n0 = idx_0.shape[0]
    n1 = idx_1.shape[0]
    n_elem0 = params_0.size // B0
    pw0 = _round_up(_round_up(n0, NW) // NW, 128)
    n1p = _round_up(n1, 128)
    pad0 = NW * pw0 - n0

    p0_phys = (params_0.reshape(B0, 256, 8, 16, 128)
               .transpose(0, 1, 3, 2, 4).reshape(B0, -1))
    p1_phys = (params_1.reshape(8, 16, 128)
               .transpose(1, 0, 2).reshape(1, -1))
    fill0 = (jnp.arange(pad0, dtype=jnp.int32) * 16) % n_elem0
    idx0p = jnp.concatenate([idx_0.astype(jnp.int32), fill0])
    idx1p = jnp.zeros((n1p,), jnp.int32).at[:n1].set(idx_1.astype(jnp.int32))

    out0p, out1p = _gather_sc(p0_phys, p1_phys, idx0p, idx1p, pw0, n1p)
    return (out0p[:, :n0], out1p[:, :n1])


# speedup vs baseline: 1.0589x; 1.0589x over previous
import functools

import jax
import jax.numpy as jnp
from jax import lax
from jax.experimental import pallas as pl
from jax.experimental.pallas import tpu as pltpu
from jax.experimental.pallas import tpu_sc as plsc

NC = 2
NS = 16
NW = NC * NS
L = 16
CH = 1024
SUB = 128


def _round_up(x, m):
    return (x + m - 1) // m * m


@functools.partial(jax.jit, static_argnums=(4, 5))
def _gather_sc(p0_lines, p1_flat, idx0p, idx1p, pw0, n1p):
    B0 = p0_lines.shape[0]
    B1 = p1_flat.shape[0]
    nch = pw0 // CH
    mesh = plsc.VectorSubcoreMesh(core_axis_name="c", subcore_axis_name="s")

    @functools.partial(
        pl.kernel,
        mesh=mesh,
        out_type=[
            jax.ShapeDtypeStruct((B0, NW * pw0), jnp.float32),
            jax.ShapeDtypeStruct((B1, n1p), jnp.float32),
        ],
        scratch_types=[
            pltpu.VMEM((pw0 + L,), jnp.int32),
            pltpu.VMEM((pw0,), jnp.int32),
            pltpu.VMEM((pw0,), jnp.int32),
            pltpu.VMEM((CH, L), jnp.float32),
            pltpu.VMEM((CH, L), jnp.float32),
            pltpu.VMEM((CH,), jnp.float32),
            pltpu.VMEM((CH,), jnp.float32),
            pltpu.SMEM((16,), jnp.int32),
            pltpu.VMEM((n1p,), jnp.int32),
            pltpu.VMEM((n1p,), jnp.float32),
            pltpu.SemaphoreType.DMA,
            pltpu.SemaphoreType.DMA,
            pltpu.SemaphoreType.DMA,
            pltpu.SemaphoreType.DMA,
        ],
        compiler_params=pltpu.CompilerParams(
            use_tc_tiling_on_sc=False, needs_layout_passes=False),
    )
    def body(p0, p1, i0, i1, out0, out1, idx_v, sel_v, ulines_v,
             lines0, lines1, pack0, pack1, ucnt_s, idx1_v, vals1_v,
             gsem0, gsem1, ssem0, ssem1):
        wid = lax.axis_index("c") * NS + lax.axis_index("s")
        base = wid * pw0
        iota = lax.iota(jnp.int32, L)
        idx_v[pl.ds(0, L)] = jnp.full((L,), -1, jnp.int32)
        pltpu.sync_copy(i0.at[pl.ds(base, pw0)], idx_v.at[pl.ds(L, pw0)])

        def dedup(s, cnt):
            ch = s >> 6
            first = (s & 63) == 0
            v = idx_v[pl.ds(L + s * L, L)]
            ln = v >> 4
            lane = v & jnp.int32(L - 1)
            pln = idx_v[pl.ds(L - 1 + s * L, L)] >> 4
            fl = (ln != pln) | ((iota == 0) & first)
            cnt0 = jnp.where(first, 0, cnt)
            cs = plsc.cumsum(fl.astype(jnp.int32))
            pos = cnt0 + cs - 1
            sel_v[pl.ds(s * L, L)] = pos * L + lane
            plsc.store_scatter(ulines_v, [pos + ch * CH], ln, mask=fl)
            cnt1 = cnt0 + jnp.max(cs)

            @pl.when((s & 63) == 63)
            def _finish_chunk():
                ucnt_s[ch] = cnt1
                rounded = (cnt1 + jnp.int32(SUB - 1)) & jnp.int32(~(SUB - 1))
                for t in range(SUB // L):
                    p = cnt1 + iota + t * L
                    plsc.store_scatter(
                        ulines_v, [ch * CH + p], iota + t * L,
                        mask=p < rounded)

            return cnt1

        lax.fori_loop(0, pw0 // L, dedup, jnp.int32(0))

        linesb = (lines0, lines1)
        packb = (pack0, pack1)
        gsems = (gsem0, gsem1)
        ssems = (ssem0, ssem1)

        def fire(ch, par):
            nsub = (ucnt_s[ch] + jnp.int32(SUB - 1)) >> 7

            def fk(k, _):
                pltpu.async_copy(
                    p0.at[cur_b].at[ulines_v.at[pl.ds(ch * CH + k * SUB, SUB)]],
                    linesb[par].at[pl.ds(k * SUB, SUB)], gsems[par])
                return 0

            lax.fori_loop(0, nsub, fk, 0)

        def drain(ch, par):
            nsub = (ucnt_s[ch] + jnp.int32(SUB - 1)) >> 7

            def dk(k, _):
                pltpu.make_async_copy(
                    p0.at[cur_b].at[ulines_v.at[pl.ds(ch * CH + k * SUB, SUB)]],
                    linesb[par].at[pl.ds(k * SUB, SUB)], gsems[par]).wait()
                return 0

            lax.fori_loop(0, nsub, dk, 0)

        def expand(ch, par):
            def eg(g, _):
                sel = sel_v[pl.ds(ch * CH + g * L, L)]
                packb[par][pl.ds(g * L, L)] = plsc.load_gather(
                    linesb[par], [sel >> 4, sel & jnp.int32(L - 1)])
                return 0

            lax.fori_loop(0, CH // L, eg, 0)

        def store_wait(ch, par):
            pltpu.make_async_copy(
                packb[par],
                out0.at[cur_b].at[pl.ds(base + ch * CH, CH)],
                ssems[par]).wait()

        def store(ch, par):
            pltpu.async_copy(
                packb[par],
                out0.at[cur_b].at[pl.ds(base + ch * CH, CH)],
                ssems[par])

        for b in range(B0):
            cur_b = b
            fire(0, 0)

            def step(ch, _):
                par = ch & 1

                def run(p):
                    @pl.when(ch + 1 < nch)
                    def _():
                        fire(ch + 1, 1 - p)
                    drain(ch, p)

                    @pl.when(ch >= 2)
                    def _():
                        store_wait(ch - 2, p)
                    expand(ch, p)
                    store(ch, p)

                @pl.when(par == 0)
                def _p0():
                    run(0)

                @pl.when(par == 1)
                def _p1():
                    run(1)

                return 0

            lax.fori_loop(0, nch, step, 0)
            store_wait(nch - 2, (nch - 2) & 1)
            store_wait(nch - 1, (nch - 1) & 1)

        @pl.when(wid < B1)
        def _small():
            pltpu.sync_copy(i1.at[pl.ds(0, n1p)], idx1_v)
            pltpu.async_copy(p1.at[wid].at[idx1_v], vals1_v, gsem0).wait()
            pltpu.sync_copy(vals1_v, out1.at[wid])

    return body(p0_lines, p1_flat, idx0p, idx1p)


def kernel(params_0, params_1, idx_0, idx_1):
    B0 = params_0.shape[0]
    n0 = idx_0.shape[0]
    n1 = idx_1.shape[0]
    n_elem0 = params_0.size // B0
    pw0 = _round_up(_round_up(n0, NW) // NW, CH)
    n1p = _round_up(n1, 128)
    pad0 = NW * pw0 - n0

    p0_lines = params_0.reshape(B0, -1, L)
    fill0 = (jnp.arange(pad0, dtype=jnp.int32) * 16) % n_elem0
    idx0p = jnp.concatenate([idx_0.astype(jnp.int32), fill0])
    idx1p = jnp.zeros((n1p,), jnp.int32).at[:n1].set(idx_1.astype(jnp.int32))

    out0p, out1p = _gather_sc(p0_lines, params_1, idx0p, idx1p, pw0, n1p)
    return (out0p[:, :n0], out1p[:, :n1])


# speedup vs baseline: 1.4552x; 1.3743x over previous
import functools

import jax
import jax.numpy as jnp
from jax import lax
from jax.experimental import pallas as pl
from jax.experimental.pallas import tpu as pltpu
from jax.experimental.pallas import tpu_sc as plsc

NC = 2
NS = 16
NW = NC * NS
L = 16
CE = 16384
NBUF = 4
BIG = 1 << 30


def _round_up(x, m):
    return (x + m - 1) // m * m


@functools.partial(jax.jit, static_argnums=(4, 5))
def _gather_sc(p0_flat, p1_flat, idx0p, idx1p, pw0, n1p):
    B0 = p0_flat.shape[0]
    B1 = p1_flat.shape[0]
    n_elem = p0_flat.shape[1]
    nchunk = n_elem // CE
    ngrp = pw0 // L
    mesh = plsc.VectorSubcoreMesh(core_axis_name="c", subcore_axis_name="s")

    @functools.partial(
        pl.kernel,
        mesh=mesh,
        out_type=[
            jax.ShapeDtypeStruct((B0, NW * pw0), jnp.float32),
            jax.ShapeDtypeStruct((B1, n1p), jnp.float32),
        ],
        scratch_types=[
            pltpu.VMEM((pw0 + L,), jnp.int32),
            pltpu.VMEM((NBUF * CE,), jnp.float32),
            pltpu.VMEM((pw0,), jnp.float32),
            pltpu.VMEM((pw0,), jnp.float32),
            pltpu.VMEM((nchunk,), jnp.int32),
            pltpu.VMEM((nchunk,), jnp.int32),
            pltpu.VMEM((n1p,), jnp.int32),
            pltpu.VMEM((n1p,), jnp.float32),
            pltpu.SemaphoreType.DMA,
            pltpu.SemaphoreType.DMA,
            pltpu.SemaphoreType.DMA,
        ],
        compiler_params=pltpu.CompilerParams(
            use_tc_tiling_on_sc=False, needs_layout_passes=False),
    )
    def body(p0, p1, i0, i1, out0, out1, idx_v, ring_v, pack0, pack1,
             gfirst_v, glast_v, idx1_v, vals1_v, gsem, ssem, s1sem):
        wid = lax.axis_index("c") * NS + lax.axis_index("s")
        base = wid * pw0
        iota = lax.iota(jnp.int32, L)
        idx_v[pl.ds(0, L)] = jnp.full((L,), -1, jnp.int32)
        pltpu.sync_copy(i0.at[pl.ds(base, pw0)], idx_v.at[pl.ds(L, pw0)])

        def sread(ref, i):
            return jnp.max(plsc.load_gather(
                ref, [jnp.full((L,), i, jnp.int32)]))

        def init_tab(t, _):
            gfirst_v[pl.ds(t * L, L)] = jnp.full((L,), BIG, jnp.int32)
            glast_v[pl.ds(t * L, L)] = jnp.full((L,), -1, jnp.int32)
            return 0

        lax.fori_loop(0, nchunk // L, init_tab, 0)

        def scan_groups(s, _):
            v = idx_v[pl.ds(L + s * L, L)]
            pv = idx_v[pl.ds(L - 1 + s * L, L)]
            c = v >> 14
            newc = c != (pv >> 14)
            ingrp = newc | (iota == 0)
            plsc.store_scatter(gfirst_v, [c], jnp.full((L,), s, jnp.int32),
                               mask=newc)
            plsc.store_scatter(glast_v, [c], jnp.full((L,), s, jnp.int32),
                               mask=ingrp)
            return 0

        lax.fori_loop(0, ngrp, scan_groups, 0)

        c_lo = sread(idx_v, L) >> 14
        c_hi = sread(idx_v, L + pw0 - 1) >> 14
        nblk = c_hi - c_lo + 1

        packs = (pack0, pack1)

        def fire(k):
            c = c_lo + k
            slot = (c & jnp.int32(NBUF - 1)) * CE
            pltpu.async_copy(
                p0.at[cur_b].at[pl.ds(c * CE, CE)],
                ring_v.at[pl.ds(slot, CE)], gsem)

        def drain(k):
            c = c_lo + k
            slot = (c & jnp.int32(NBUF - 1)) * CE
            pltpu.make_async_copy(
                p0.at[cur_b].at[pl.ds(c * CE, CE)],
                ring_v.at[pl.ds(slot, CE)], gsem).wait()

        for b in range(B0):
            cur_b = b
            pack_v = packs[b & 1]
            fire(0)

            @pl.when(nblk > 1)
            def _pro():
                fire(1)

            if b >= 2:
                pltpu.make_async_copy(
                    pack_v, out0.at[b - 2].at[pl.ds(base, pw0)], ssem).wait()

            def step(k, _):
                @pl.when(k + 2 < nblk)
                def _():
                    fire(k + 2)
                drain(k)
                c = c_lo + k
                glo = sread(gfirst_v, c)
                ghi = sread(glast_v, c)
                glo = jnp.minimum(glo, ghi + 1)

                def expand(g, _):
                    v = idx_v[pl.ds(L + g * L, L)]
                    m = (v >> 14) == c
                    addr = ((v >> 14) & jnp.int32(NBUF - 1)) * CE \
                        + (v & jnp.int32(CE - 1))
                    vals = plsc.load_gather(ring_v, [addr], mask=m)
                    prev = pack_v[pl.ds(g * L, L)]
                    pack_v[pl.ds(g * L, L)] = jnp.where(m, vals, prev)
                    return 0

                lax.fori_loop(glo, ghi + 1, expand, 0)
                return 0

            lax.fori_loop(0, nblk, step, 0)
            pltpu.async_copy(pack_v, out0.at[b].at[pl.ds(base, pw0)], ssem)

        pltpu.make_async_copy(
            packs[(B0 - 2) & 1],
            out0.at[B0 - 2].at[pl.ds(base, pw0)], ssem).wait()
        pltpu.make_async_copy(
            packs[(B0 - 1) & 1],
            out0.at[B0 - 1].at[pl.ds(base, pw0)], ssem).wait()

        @pl.when(wid < B1)
        def _small():
            pltpu.sync_copy(i1.at[pl.ds(0, n1p)], idx1_v)
            pltpu.async_copy(p1.at[wid].at[idx1_v], vals1_v, s1sem).wait()
            pltpu.sync_copy(vals1_v, out1.at[wid])

    return body(p0_flat, p1_flat, idx0p, idx1p)


def kernel(params_0, params_1, idx_0, idx_1):
    B0 = params_0.shape[0]
    n0 = idx_0.shape[0]
    n1 = idx_1.shape[0]
    pw0 = _round_up(_round_up(n0, NW) // NW, 128)
    n1p = _round_up(n1, 128)
    pad0 = NW * pw0 - n0

    p0_flat = params_0.reshape(B0, -1)
    fill0 = jnp.full((pad0,), idx_0[-1], jnp.int32)
    idx0p = jnp.concatenate([idx_0.astype(jnp.int32), fill0])
    idx1p = jnp.zeros((n1p,), jnp.int32).at[:n1].set(idx_1.astype(jnp.int32))

    out0p, out1p = _gather_sc(p0_flat, params_1, idx0p, idx1p, pw0, n1p)
    return (out0p[:, :n0], out1p[:, :n1])


# speedup vs baseline: 1.5386x; 1.0573x over previous
import functools

import jax
import jax.numpy as jnp
from jax import lax
from jax.experimental import pallas as pl
from jax.experimental.pallas import tpu as pltpu
from jax.experimental.pallas import tpu_sc as plsc

NC = 2
NS = 16
NW = NC * NS
L = 16
CE = 16384
NBUF = 4
BIG = 1 << 30


def _round_up(x, m):
    return (x + m - 1) // m * m


@functools.partial(jax.jit, static_argnums=(4, 5))
def _gather_sc(p0_flat, p1_flat, idx0p, idx1p, pw0, n1p):
    B0 = p0_flat.shape[0]
    B1 = p1_flat.shape[0]
    n_elem = p0_flat.shape[1]
    nchunk = n_elem // CE
    ngrp = pw0 // L
    mesh = plsc.VectorSubcoreMesh(core_axis_name="c", subcore_axis_name="s")

    @functools.partial(
        pl.kernel,
        mesh=mesh,
        out_type=[
            jax.ShapeDtypeStruct((B0, NW * pw0), jnp.float32),
            jax.ShapeDtypeStruct((B1, n1p), jnp.float32),
        ],
        scratch_types=[
            pltpu.VMEM((pw0 + L,), jnp.int32),
            pltpu.VMEM((NBUF * CE,), jnp.float32),
            pltpu.VMEM((pw0,), jnp.float32),
            pltpu.VMEM((pw0,), jnp.float32),
            pltpu.VMEM((nchunk,), jnp.int32),
            pltpu.VMEM((nchunk,), jnp.int32),
            pltpu.VMEM((n1p,), jnp.int32),
            pltpu.VMEM((n1p,), jnp.float32),
            pltpu.SemaphoreType.DMA,
            pltpu.SemaphoreType.DMA,
            pltpu.SemaphoreType.DMA,
        ],
        compiler_params=pltpu.CompilerParams(
            use_tc_tiling_on_sc=False, needs_layout_passes=False),
    )
    def body(p0, p1, i0, i1, out0, out1, idx_v, ring_v, pack0, pack1,
             gfirst_v, glast_v, idx1_v, vals1_v, gsem, ssem, s1sem):
        wid = lax.axis_index("c") * NS + lax.axis_index("s")
        base = wid * pw0
        iota = lax.iota(jnp.int32, L)
        idx_v[pl.ds(0, L)] = jnp.full((L,), -1, jnp.int32)
        pltpu.sync_copy(i0.at[pl.ds(base, pw0)], idx_v.at[pl.ds(L, pw0)])

        def sread(ref, i):
            return jnp.max(plsc.load_gather(
                ref, [jnp.full((L,), i, jnp.int32)]))

        def init_tab(t, _):
            gfirst_v[pl.ds(t * L, L)] = jnp.full((L,), BIG, jnp.int32)
            glast_v[pl.ds(t * L, L)] = jnp.full((L,), -1, jnp.int32)
            return 0

        lax.fori_loop(0, nchunk // L, init_tab, 0)

        def scan_groups(s, _):
            v = idx_v[pl.ds(L + s * L, L)]
            pv = idx_v[pl.ds(L - 1 + s * L, L)]
            c = v >> 14
            newc = c != (pv >> 14)
            ingrp = newc | (iota == 0)
            plsc.store_scatter(gfirst_v, [c], jnp.full((L,), s, jnp.int32),
                               mask=newc)
            plsc.store_scatter(glast_v, [c], jnp.full((L,), s, jnp.int32),
                               mask=ingrp)
            return 0

        lax.fori_loop(0, ngrp, scan_groups, 0)

        c_lo = sread(idx_v, L) >> 14
        c_hi = sread(idx_v, L + pw0 - 1) >> 14
        nblk = c_hi - c_lo + 1

        packs = (pack0, pack1)

        def fire(k):
            c = c_lo + k
            slot = (c & jnp.int32(NBUF - 1)) * CE
            pltpu.async_copy(
                p0.at[cur_b].at[pl.ds(c * CE, CE)],
                ring_v.at[pl.ds(slot, CE)], gsem)

        def drain(k):
            c = c_lo + k
            slot = (c & jnp.int32(NBUF - 1)) * CE
            pltpu.make_async_copy(
                p0.at[cur_b].at[pl.ds(c * CE, CE)],
                ring_v.at[pl.ds(slot, CE)], gsem).wait()

        for b in range(B0):
            cur_b = b
            pack_v = packs[b & 1]
            fire(0)

            @pl.when(nblk > 1)
            def _pro():
                fire(1)

            if b >= 2:
                pltpu.make_async_copy(
                    pack_v, out0.at[b - 2].at[pl.ds(base, pw0)], ssem).wait()

            def step(k, _):
                @pl.when(k + 2 < nblk)
                def _():
                    fire(k + 2)
                drain(k)
                c = c_lo + k
                glo = sread(gfirst_v, c)
                ghi = sread(glast_v, c)
                glo = jnp.minimum(glo, ghi + 1)

                def masked_expand(g):
                    v = idx_v[pl.ds(L + g * L, L)]
                    m = (v >> 14) == c
                    vals = plsc.load_gather(
                        ring_v, [v & jnp.int32(NBUF * CE - 1)], mask=m)
                    prev = pack_v[pl.ds(g * L, L)]
                    pack_v[pl.ds(g * L, L)] = jnp.where(m, vals, prev)

                masked_expand(glo)

                @pl.when(ghi > glo)
                def _():
                    masked_expand(ghi)

                def fast_expand(g, _):
                    v = idx_v[pl.ds(L + g * L, L)]
                    pack_v[pl.ds(g * L, L)] = plsc.load_gather(
                        ring_v, [v & jnp.int32(NBUF * CE - 1)])
                    return 0

                lax.fori_loop(glo + 1, jnp.maximum(ghi, glo + 1),
                              fast_expand, 0)
                return 0

            lax.fori_loop(0, nblk, step, 0)
            pltpu.async_copy(pack_v, out0.at[b].at[pl.ds(base, pw0)], ssem)

        pltpu.make_async_copy(
            packs[(B0 - 2) & 1],
            out0.at[B0 - 2].at[pl.ds(base, pw0)], ssem).wait()
        pltpu.make_async_copy(
            packs[(B0 - 1) & 1],
            out0.at[B0 - 1].at[pl.ds(base, pw0)], ssem).wait()

        @pl.when(wid < B1)
        def _small():
            pltpu.sync_copy(i1.at[pl.ds(0, n1p)], idx1_v)
            pltpu.async_copy(p1.at[wid].at[idx1_v], vals1_v, s1sem).wait()
            pltpu.sync_copy(vals1_v, out1.at[wid])

    return body(p0_flat, p1_flat, idx0p, idx1p)


def kernel(params_0, params_1, idx_0, idx_1):
    B0 = params_0.shape[0]
    n0 = idx_0.shape[0]
    n1 = idx_1.shape[0]
    pw0 = _round_up(_round_up(n0, NW) // NW, 128)
    n1p = _round_up(n1, 128)
    pad0 = NW * pw0 - n0

    p0_flat = params_0.reshape(B0, -1)
    fill0 = jnp.full((pad0,), idx_0[-1], jnp.int32)
    idx0p = jnp.concatenate([idx_0.astype(jnp.int32), fill0])
    idx1p = jnp.zeros((n1p,), jnp.int32).at[:n1].set(idx_1.astype(jnp.int32))

    out0p, out1p = _gather_sc(p0_flat, params_1, idx0p, idx1p, pw0, n1p)
    return (out0p[:, :n0], out1p[:, :n1])


# speedup vs baseline: 2.5435x; 1.6531x over previous
import functools

import jax
import jax.numpy as jnp
from jax import lax
from jax.experimental import pallas as pl
from jax.experimental.pallas import tpu as pltpu
from jax.experimental.pallas import tpu_sc as plsc

NC = 2
NS = 16
NW = NC * NS
L = 16
CE = 16384
NBUF = 4
BIG = 1 << 30


def _round_up(x, m):
    return (x + m - 1) // m * m


@functools.partial(jax.jit, static_argnums=(4, 5))
def _gather_sc(p0_br, p1, idx0p, idx1p, pw0, n1p):
    B0 = p0_br.shape[0]
    B1 = p1.shape[0]
    nchunk = p0_br.shape[1]
    ncol = p0_br.shape[3]
    ngrp = pw0 // L
    mesh = plsc.VectorSubcoreMesh(core_axis_name="c", subcore_axis_name="s")

    @functools.partial(
        pl.kernel,
        mesh=mesh,
        out_type=[
            jax.ShapeDtypeStruct((B0, NW * pw0), jnp.float32),
            jax.ShapeDtypeStruct((B1, n1p), jnp.float32),
        ],
        scratch_types=[
            pltpu.VMEM((pw0 + 128,), jnp.int32),
            pltpu.VMEM((NBUF * 8, ncol), jnp.float32),
            pltpu.VMEM((pw0,), jnp.float32),
            pltpu.VMEM((pw0,), jnp.float32),
            pltpu.VMEM((nchunk,), jnp.int32),
            pltpu.VMEM((nchunk,), jnp.int32),
            pltpu.VMEM((B1, ncol), jnp.float32),
            pltpu.VMEM((n1p,), jnp.int32),
            pltpu.VMEM((n1p,), jnp.float32),
            pltpu.SemaphoreType.DMA,
            pltpu.SemaphoreType.DMA,
        ],
        compiler_params=pltpu.CompilerParams(
            use_tc_tiling_on_sc=True, needs_layout_passes=False),
    )
    def body(p0, p1r, i0, i1, out0, out1, idx_v, ring_v, pack0, pack1,
             gfirst_v, glast_v, p1_v, idx1_v, vals1_v, gsem, ssem):
        wid = lax.axis_index("c") * NS + lax.axis_index("s")
        base = wid * pw0
        iota = lax.iota(jnp.int32, L)
        idx_v[pl.ds(112, L)] = jnp.full((L,), -1, jnp.int32)
        pltpu.sync_copy(i0.at[pl.ds(base, pw0)], idx_v.at[pl.ds(128, pw0)])

        def sread(ref, i):
            return jnp.max(plsc.load_gather(
                ref, [jnp.full((L,), i, jnp.int32)]))

        def init_tab(t, _):
            gfirst_v[pl.ds(t * L, L)] = jnp.full((L,), BIG, jnp.int32)
            glast_v[pl.ds(t * L, L)] = jnp.full((L,), -1, jnp.int32)
            return 0

        lax.fori_loop(0, nchunk // L, init_tab, 0)

        def scan_groups(s, _):
            v = idx_v[pl.ds(128 + s * L, L)]
            pv = idx_v[pl.ds(127 + s * L, L)]
            c = v >> 14
            newc = c != (pv >> 14)
            ingrp = newc | (iota == 0)
            plsc.store_scatter(gfirst_v, [c], jnp.full((L,), s, jnp.int32),
                               mask=newc)
            plsc.store_scatter(glast_v, [c], jnp.full((L,), s, jnp.int32),
                               mask=ingrp)
            return 0

        lax.fori_loop(0, ngrp, scan_groups, 0)

        c_lo = sread(idx_v, 128) >> 14
        c_hi = sread(idx_v, 128 + pw0 - 1) >> 14
        nblk = c_hi - c_lo + 1

        packs = (pack0, pack1)

        def fire(k):
            c = c_lo + k
            slot = (c & jnp.int32(NBUF - 1)) * 8
            pltpu.async_copy(
                p0.at[cur_b].at[c],
                ring_v.at[pl.ds(slot, 8)], gsem)

        def drain(k):
            c = c_lo + k
            slot = (c & jnp.int32(NBUF - 1)) * 8
            pltpu.make_async_copy(
                p0.at[cur_b].at[c],
                ring_v.at[pl.ds(slot, 8)], gsem).wait()

        for b in range(B0):
            cur_b = b
            pack_v = packs[b & 1]
            fire(0)

            @pl.when(nblk > 1)
            def _pro():
                fire(1)

            if b >= 2:
                pltpu.make_async_copy(
                    pack_v, out0.at[b - 2].at[pl.ds(base, pw0)], ssem).wait()

            def step(k, _):
                @pl.when(k + 2 < nblk)
                def _():
                    fire(k + 2)
                drain(k)
                c = c_lo + k
                glo = sread(gfirst_v, c)
                ghi = sread(glast_v, c)
                glo = jnp.minimum(glo, ghi + 1)

                def addrs(v):
                    return [(v >> 11) & jnp.int32(NBUF * 8 - 1),
                            v & jnp.int32(ncol - 1)]

                def masked_expand(g):
                    v = idx_v[pl.ds(128 + g * L, L)]
                    m = (v >> 14) == c
                    vals = plsc.load_gather(ring_v, addrs(v), mask=m)
                    prev = pack_v[pl.ds(g * L, L)]
                    pack_v[pl.ds(g * L, L)] = jnp.where(m, vals, prev)

                masked_expand(glo)

                @pl.when(ghi > glo)
                def _():
                    masked_expand(ghi)

                def fast_expand(g, _):
                    v = idx_v[pl.ds(128 + g * L, L)]
                    pack_v[pl.ds(g * L, L)] = plsc.load_gather(
                        ring_v, addrs(v))
                    return 0

                lax.fori_loop(glo + 1, jnp.maximum(ghi, glo + 1),
                              fast_expand, 0)
                return 0

            lax.fori_loop(0, nblk, step, 0)
            pltpu.async_copy(pack_v, out0.at[b].at[pl.ds(base, pw0)], ssem)

        pltpu.make_async_copy(
            packs[(B0 - 2) & 1],
            out0.at[B0 - 2].at[pl.ds(base, pw0)], ssem).wait()
        pltpu.make_async_copy(
            packs[(B0 - 1) & 1],
            out0.at[B0 - 1].at[pl.ds(base, pw0)], ssem).wait()

        @pl.when(wid < B1)
        def _small():
            pltpu.sync_copy(i1.at[pl.ds(0, n1p)], idx1_v)
            pltpu.sync_copy(p1r, p1_v)

            def small_g(t, _):
                j = idx1_v[pl.ds(t * L, L)]
                vals1_v[pl.ds(t * L, L)] = plsc.load_gather(
                    p1_v, [jnp.full((L,), wid, jnp.int32), j])
                return 0

            lax.fori_loop(0, n1p // L, small_g, 0)
            pltpu.sync_copy(vals1_v, out1.at[wid])

    return body(p0_br, p1, idx0p, idx1p)


def kernel(params_0, params_1, idx_0, idx_1):
    B0 = params_0.shape[0]
    n0 = idx_0.shape[0]
    n1 = idx_1.shape[0]
    pw0 = _round_up(_round_up(n0, NW) // NW, 128)
    n1p = _round_up(n1, 128)
    pad0 = NW * pw0 - n0

    p0_br = params_0.reshape(B0, 256, 8, 2048)
    fill0 = jnp.full((pad0,), idx_0[-1], jnp.int32)
    idx0p = jnp.concatenate([idx_0.astype(jnp.int32), fill0])
    idx1p = jnp.zeros((n1p,), jnp.int32).at[:n1].set(idx_1.astype(jnp.int32))

    out0p, out1p = _gather_sc(p0_br, params_1, idx0p, idx1p, pw0, n1p)
    return (out0p[:, :n0], out1p[:, :n1])


# speedup vs baseline: 2.5649x; 1.0084x over previous
import functools

import jax
import jax.numpy as jnp
from jax import lax
from jax.experimental import pallas as pl
from jax.experimental.pallas import tpu as pltpu
from jax.experimental.pallas import tpu_sc as plsc

NC = 2
NS = 16
NW = NC * NS
L = 16
CE = 16384
NBUF = 4
BIG = 1 << 30


def _round_up(x, m):
    return (x + m - 1) // m * m


@functools.partial(jax.jit, static_argnums=(4, 5, 6))
def _gather_sc(p0_br, p1, idx0, idx1, pw0, n1p, n0):
    B0 = p0_br.shape[0]
    B1 = p1.shape[0]
    last0 = n0 - (NW - 1) * pw0
    nchunk = p0_br.shape[1]
    ncol = p0_br.shape[3]
    ngrp = pw0 // L
    mesh = plsc.VectorSubcoreMesh(core_axis_name="c", subcore_axis_name="s")

    @functools.partial(
        pl.kernel,
        mesh=mesh,
        out_type=[
            jax.ShapeDtypeStruct((B0, NW * pw0), jnp.float32),
            jax.ShapeDtypeStruct((B1, n1p), jnp.float32),
        ],
        scratch_types=[
            pltpu.VMEM((pw0 + 144,), jnp.int32),
            pltpu.VMEM((NBUF * 8, ncol), jnp.float32),
            pltpu.VMEM((pw0,), jnp.float32),
            pltpu.VMEM((pw0,), jnp.float32),
            pltpu.VMEM((nchunk,), jnp.int32),
            pltpu.VMEM((nchunk,), jnp.int32),
            pltpu.VMEM((B1, ncol), jnp.float32),
            pltpu.VMEM((n1p,), jnp.int32),
            pltpu.VMEM((n1p,), jnp.float32),
            pltpu.SemaphoreType.DMA,
            pltpu.SemaphoreType.DMA,
        ],
        compiler_params=pltpu.CompilerParams(
            use_tc_tiling_on_sc=True, needs_layout_passes=False),
    )
    def body(p0, p1r, i0, i1, out0, out1, idx_v, ring_v, pack0, pack1,
             gfirst_v, glast_v, p1_v, idx1_v, vals1_v, gsem, ssem):
        wid = lax.axis_index("c") * NS + lax.axis_index("s")
        base = wid * pw0
        iota = lax.iota(jnp.int32, L)
        idx_v[pl.ds(112, L)] = jnp.full((L,), -1, jnp.int32)

        def sread(ref, i):
            return jnp.max(plsc.load_gather(
                ref, [jnp.full((L,), i, jnp.int32)]))

        @pl.when(wid < NW - 1)
        def _stage_full():
            pltpu.sync_copy(i0.at[pl.ds(base, pw0)], idx_v.at[pl.ds(128, pw0)])

        @pl.when(wid == NW - 1)
        def _stage_tail():
            last0r = _round_up(last0, 128)
            pltpu.sync_copy(i0.at[pl.ds(base, last0r)],
                            idx_v.at[pl.ds(128, last0r)])
            lastv = sread(idx_v, 128 + last0 - 1)
            fillv = jnp.full((L,), 0, jnp.int32) + lastv

            def fill(t, _):
                idx_v[pl.ds(128 + last0 + t * L, L)] = fillv
                return 0

            lax.fori_loop(0, (pw0 - last0 + L - 1) // L, fill, 0)

        def init_tab(t, _):
            gfirst_v[pl.ds(t * L, L)] = jnp.full((L,), BIG, jnp.int32)
            glast_v[pl.ds(t * L, L)] = jnp.full((L,), -1, jnp.int32)
            return 0

        lax.fori_loop(0, nchunk // L, init_tab, 0)

        def scan_groups(s, _):
            v = idx_v[pl.ds(128 + s * L, L)]
            pv = idx_v[pl.ds(127 + s * L, L)]
            c = v >> 14
            newc = c != (pv >> 14)
            ingrp = newc | (iota == 0)
            plsc.store_scatter(gfirst_v, [c], jnp.full((L,), s, jnp.int32),
                               mask=newc)
            plsc.store_scatter(glast_v, [c], jnp.full((L,), s, jnp.int32),
                               mask=ingrp)
            return 0

        lax.fori_loop(0, ngrp, scan_groups, 0)

        c_lo = sread(idx_v, 128) >> 14
        c_hi = sread(idx_v, 128 + pw0 - 1) >> 14
        nblk = c_hi - c_lo + 1

        packs = (pack0, pack1)

        def store_out(b_, pack, wait):
            cp = pltpu.make_async_copy(
                pack, out0.at[b_].at[pl.ds(base, pw0)], ssem)
            cp.wait() if wait else cp.start()

        def fire(k):
            c = c_lo + k
            slot = (c & jnp.int32(NBUF - 1)) * 8
            pltpu.async_copy(
                p0.at[cur_b].at[c],
                ring_v.at[pl.ds(slot, 8)], gsem)

        def drain(k):
            c = c_lo + k
            slot = (c & jnp.int32(NBUF - 1)) * 8
            pltpu.make_async_copy(
                p0.at[cur_b].at[c],
                ring_v.at[pl.ds(slot, 8)], gsem).wait()

        for b in range(B0):
            cur_b = b
            pack_v = packs[b & 1]
            fire(0)

            @pl.when(nblk > 1)
            def _pro():
                fire(1)

            if b >= 2:
                store_out(b - 2, pack_v, wait=True)

            def step(k, _):
                @pl.when(k + 2 < nblk)
                def _():
                    fire(k + 2)
                drain(k)
                c = c_lo + k
                glo = sread(gfirst_v, c)
                ghi = sread(glast_v, c)
                glo = jnp.minimum(glo, ghi + 1)

                def addrs(v):
                    return [(v >> 11) & jnp.int32(NBUF * 8 - 1),
                            v & jnp.int32(ncol - 1)]

                def masked_expand(g):
                    v = idx_v[pl.ds(128 + g * L, L)]
                    m = (v >> 14) == c
                    vals = plsc.load_gather(ring_v, addrs(v), mask=m)
                    prev = pack_v[pl.ds(g * L, L)]
                    pack_v[pl.ds(g * L, L)] = jnp.where(m, vals, prev)

                masked_expand(glo)

                @pl.when(ghi > glo)
                def _():
                    masked_expand(ghi)

                def fast_expand(g, _):
                    v = idx_v[pl.ds(128 + g * L, L)]
                    pack_v[pl.ds(g * L, L)] = plsc.load_gather(
                        ring_v, addrs(v))
                    return 0

                lax.fori_loop(glo + 1, jnp.maximum(ghi, glo + 1),
                              fast_expand, 0)
                return 0

            lax.fori_loop(0, nblk, step, 0)
            store_out(b, pack_v, wait=False)

        store_out(B0 - 2, packs[(B0 - 2) & 1], wait=True)
        store_out(B0 - 1, packs[(B0 - 1) & 1], wait=True)

        @pl.when(wid < B1)
        def _small():
            pltpu.sync_copy(i1.at[pl.ds(0, n1p)], idx1_v)
            pltpu.sync_copy(p1r, p1_v)

            def small_g(t, _):
                j = idx1_v[pl.ds(t * L, L)]
                vals1_v[pl.ds(t * L, L)] = plsc.load_gather(
                    p1_v, [jnp.full((L,), wid, jnp.int32), j])
                return 0

            lax.fori_loop(0, n1p // L, small_g, 0)
            pltpu.sync_copy(vals1_v, out1.at[wid])

    return body(p0_br, p1, idx0, idx1)


def kernel(params_0, params_1, idx_0, idx_1):
    B0 = params_0.shape[0]
    n0 = idx_0.shape[0]
    n1 = idx_1.shape[0]
    pw0 = _round_up(_round_up(n0, NW) // NW, 128)
    n1p = _round_up(n1, 128)

    p0_br = params_0.reshape(B0, 256, 8, 2048)
    idx0p = jnp.zeros((_round_up(n0, 128),), jnp.int32).at[:n0].set(
        idx_0.astype(jnp.int32))
    idx1p = jnp.zeros((n1p,), jnp.int32).at[:n1].set(idx_1.astype(jnp.int32))
    out0p, out1p = _gather_sc(p0_br, params_1, idx0p, idx1p, pw0, n1p, n0)
    return (out0p[:, :n0], out1p[:, :n1])


# speedup vs baseline: 2.5735x; 1.0033x over previous
import functools

import jax
import jax.numpy as jnp
from jax import lax
from jax.experimental import pallas as pl
from jax.experimental.pallas import tpu as pltpu
from jax.experimental.pallas import tpu_sc as plsc

NC = 2
NS = 16
NW = NC * NS
L = 16
CE = 16384
NBUF = 4
BIG = 1 << 30


def _round_up(x, m):
    return (x + m - 1) // m * m


@functools.partial(jax.jit, static_argnums=(4, 5, 6))
def _gather_sc(p0_br, p1, idx0, idx1, pw0, n1p, n0):
    B0 = p0_br.shape[0]
    B1 = p1.shape[0]
    last0 = n0 - (NW - 1) * pw0
    nchunk = p0_br.shape[1]
    ncol = p0_br.shape[3]
    ngrp = pw0 // L
    mesh = plsc.VectorSubcoreMesh(core_axis_name="c", subcore_axis_name="s")

    @functools.partial(
        pl.kernel,
        mesh=mesh,
        out_type=[
            jax.ShapeDtypeStruct((B0, NW * pw0), jnp.float32),
            jax.ShapeDtypeStruct((B1, n1p), jnp.float32),
        ],
        scratch_types=[
            pltpu.VMEM((pw0 + 144,), jnp.int32),
            pltpu.VMEM((NBUF * 8, ncol), jnp.float32),
            pltpu.VMEM((pw0,), jnp.float32),
            pltpu.VMEM((pw0,), jnp.float32),
            pltpu.VMEM((nchunk,), jnp.int32),
            pltpu.VMEM((nchunk,), jnp.int32),
            pltpu.VMEM((B1, ncol), jnp.float32),
            pltpu.VMEM((n1p,), jnp.int32),
            pltpu.VMEM((n1p,), jnp.float32),
            pltpu.SemaphoreType.DMA,
            pltpu.SemaphoreType.DMA,
        ],
        compiler_params=pltpu.CompilerParams(
            use_tc_tiling_on_sc=True, needs_layout_passes=False),
    )
    def body(p0, p1r, i0, i1, out0, out1, idx_v, ring_v, pack0, pack1,
             gfirst_v, glast_v, p1_v, idx1_v, vals1_v, gsem, ssem):
        wid = lax.axis_index("c") * NS + lax.axis_index("s")
        base = wid * pw0
        iota = lax.iota(jnp.int32, L)
        idx_v[pl.ds(112, L)] = jnp.full((L,), -1, jnp.int32)

        def sread(ref, i):
            return jnp.max(plsc.load_gather(
                ref, [jnp.full((L,), i, jnp.int32)]))

        @pl.when(wid < NW - 1)
        def _stage_full():
            pltpu.sync_copy(i0.at[pl.ds(base, pw0)], idx_v.at[pl.ds(128, pw0)])

        @pl.when(wid == NW - 1)
        def _stage_tail():
            last0r = _round_up(last0, 128)
            pltpu.sync_copy(i0.at[pl.ds(base, last0r)],
                            idx_v.at[pl.ds(128, last0r)])
            lastv = sread(idx_v, 128 + last0 - 1)
            fillv = jnp.full((L,), 0, jnp.int32) + lastv

            def fill(t, _):
                idx_v[pl.ds(128 + last0 + t * L, L)] = fillv
                return 0

            lax.fori_loop(0, (pw0 - last0 + L - 1) // L, fill, 0)

        def init_tab(t, _):
            gfirst_v[pl.ds(t * L, L)] = jnp.full((L,), BIG, jnp.int32)
            glast_v[pl.ds(t * L, L)] = jnp.full((L,), -1, jnp.int32)
            return 0

        lax.fori_loop(0, nchunk // L, init_tab, 0)

        def scan_groups(s, _):
            v = idx_v[pl.ds(128 + s * L, L)]
            pv = idx_v[pl.ds(127 + s * L, L)]
            c = v >> 14
            newc = c != (pv >> 14)
            ingrp = newc | (iota == 0)
            plsc.store_scatter(gfirst_v, [c], jnp.full((L,), s, jnp.int32),
                               mask=newc)
            plsc.store_scatter(glast_v, [c], jnp.full((L,), s, jnp.int32),
                               mask=ingrp)
            return 0

        lax.fori_loop(0, ngrp, scan_groups, 0)

        c_lo = sread(idx_v, 128) >> 14
        c_hi = sread(idx_v, 128 + pw0 - 1) >> 14
        nblk = c_hi - c_lo + 1

        packs = (pack0, pack1)

        def store_out(b_, pack, wait):
            cp = pltpu.make_async_copy(
                pack, out0.at[b_].at[pl.ds(base, pw0)], ssem)
            cp.wait() if wait else cp.start()

        def fire(k):
            c = c_lo + k
            slot = (c & jnp.int32(NBUF - 1)) * 8
            pltpu.async_copy(
                p0.at[cur_b].at[c],
                ring_v.at[pl.ds(slot, 8)], gsem)

        def drain(k):
            c = c_lo + k
            slot = (c & jnp.int32(NBUF - 1)) * 8
            pltpu.make_async_copy(
                p0.at[cur_b].at[c],
                ring_v.at[pl.ds(slot, 8)], gsem).wait()

        for b in range(B0):
            cur_b = b
            pack_v = packs[b & 1]
            fire(0)

            @pl.when(nblk > 1)
            def _pro():
                fire(1)

            @pl.when(nblk > 2)
            def _pro2():
                fire(2)

            if b >= 2:
                store_out(b - 2, pack_v, wait=True)

            def step(k, _):
                @pl.when(k + 3 < nblk)
                def _():
                    fire(k + 3)
                c = c_lo + k
                glo = sread(gfirst_v, c)
                ghi = sread(glast_v, c)
                glo = jnp.minimum(glo, ghi + 1)
                drain(k)

                def addrs(v):
                    return [(v >> 11) & jnp.int32(NBUF * 8 - 1),
                            v & jnp.int32(ncol - 1)]

                def masked_expand(g):
                    v = idx_v[pl.ds(128 + g * L, L)]
                    m = (v >> 14) == c
                    vals = plsc.load_gather(ring_v, addrs(v), mask=m)
                    prev = pack_v[pl.ds(g * L, L)]
                    pack_v[pl.ds(g * L, L)] = jnp.where(m, vals, prev)

                masked_expand(glo)

                @pl.when(ghi > glo)
                def _():
                    masked_expand(ghi)

                def fast_expand(g, _):
                    v = idx_v[pl.ds(128 + g * L, L)]
                    pack_v[pl.ds(g * L, L)] = plsc.load_gather(
                        ring_v, addrs(v))
                    return 0

                lax.fori_loop(glo + 1, jnp.maximum(ghi, glo + 1),
                              fast_expand, 0)
                return 0

            lax.fori_loop(0, nblk, step, 0)
            store_out(b, pack_v, wait=False)

        store_out(B0 - 2, packs[(B0 - 2) & 1], wait=True)
        store_out(B0 - 1, packs[(B0 - 1) & 1], wait=True)

        @pl.when(wid < B1)
        def _small():
            pltpu.sync_copy(i1.at[pl.ds(0, n1p)], idx1_v)
            pltpu.sync_copy(p1r, p1_v)

            def small_g(t, _):
                j = idx1_v[pl.ds(t * L, L)]
                vals1_v[pl.ds(t * L, L)] = plsc.load_gather(
                    p1_v, [jnp.full((L,), wid, jnp.int32), j])
                return 0

            lax.fori_loop(0, n1p // L, small_g, 0)
            pltpu.sync_copy(vals1_v, out1.at[wid])

    return body(p0_br, p1, idx0, idx1)


def kernel(params_0, params_1, idx_0, idx_1):
    B0 = params_0.shape[0]
    n0 = idx_0.shape[0]
    n1 = idx_1.shape[0]
    pw0 = _round_up(_round_up(n0, NW) // NW, 128)
    n1p = _round_up(n1, 128)

    p0_br = params_0.reshape(B0, 256, 8, 2048)
    idx0p = jnp.zeros((_round_up(n0, 128),), jnp.int32).at[:n0].set(
        idx_0.astype(jnp.int32))
    idx1p = jnp.zeros((n1p,), jnp.int32).at[:n1].set(idx_1.astype(jnp.int32))
    out0p, out1p = _gather_sc(p0_br, params_1, idx0p, idx1p, pw0, n1p, n0)
    return (out0p[:, :n0], out1p[:, :n1])
